# baseline (speedup 1.0000x reference)
import functools

import jax
import jax.numpy as jnp
import ml_dtypes
import numpy as np
from jax.experimental import pallas as pl
from jax.experimental.pallas import tpu as pltpu

SEG = 896


def _bottleneck_body(x_ref, w12_ref, w3_ref, b_ref, p_ref, o_ref, *,
                     H, W, G, Cin, C2):
    HW = H * W
    L = G * SEG
    w1_v = w12_ref[:, :Cin]
    w2_v = w12_ref[:, Cin:]
    b1_v = b_ref[:C2]
    b2_v = b_ref[C2:2 * C2]
    b3_v = b_ref[2 * C2:]
    lane = jax.lax.broadcasted_iota(jnp.int32, (1, L), 1)
    pos = lane % SEG
    wmod = pos % W
    seg_m = (pos < HW).astype(jnp.bfloat16)
    m_left = (wmod >= 1).astype(jnp.bfloat16)
    m_right = (wmod <= W - 2).astype(jnp.bfloat16)

    segs = [jnp.pad(x_ref[g].astype(jnp.bfloat16), ((0, 0), (0, SEG - HW)))
            for g in range(G)]
    x2 = jnp.concatenate(segs, axis=1)

    h1 = jnp.dot(w1_v, x2, preferred_element_type=jnp.float32)
    h1 = jnp.maximum(h1 + b1_v, 0.0)
    h1b = h1.astype(jnp.bfloat16) * seg_m

    hp1 = jnp.pad(h1b, ((0, 0), (1, 1)))
    base = jnp.concatenate(
        [hp1[:, 0:L] * m_left, h1b, hp1[:, 2:2 + L] * m_right],
        axis=0)
    bp = jnp.pad(base, ((0, 0), (W, W)))
    pat = jnp.concatenate(
        [bp[:, 0:L], base, bp[:, 2 * W:2 * W + L]], axis=0)
    h2 = jnp.dot(w2_v, pat, preferred_element_type=jnp.float32)
    h2 = jnp.maximum(h2 + b2_v, 0.0)
    h2b = h2.astype(jnp.bfloat16)

    CR = C2 + Cin
    big = jnp.concatenate(
        [h2b[:, g * SEG: g * SEG + HW] if k == 0 else
         x2[:, g * SEG: g * SEG + HW]
         for g in range(G) for k in (0, 1)], axis=0)
    po = jnp.dot(big, p_ref[...], preferred_element_type=jnp.float32)
    cat = jnp.concatenate(
        [po[g * CR:(g + 1) * CR].astype(jnp.bfloat16) for g in range(G)],
        axis=1)
    y = jnp.dot(w3_ref[...], cat, preferred_element_type=jnp.float32)
    y = jnp.maximum(y + b3_v, 0.0)
    HoWo = o_ref.shape[-1]
    for g in range(G):
        o_ref[g] = y[:, g * 256: g * 256 + HoWo]


def kernel(x, w1, b1, w2, b2, w3, wd, b3):
    N, Cin, H, W = x.shape
    C2 = w1.shape[1]
    Cout = w3.shape[1]
    K = 2
    Ho, Wo = H // K, W // K
    HW, HoWo = H * W, Ho * Wo

    w12 = jnp.concatenate(
        [w1.T, w2.transpose(2, 0, 1).reshape(C2, 9 * C2)],
        axis=1).astype(jnp.bfloat16)
    w3t = jnp.concatenate([w3, wd], axis=0).T.astype(jnp.bfloat16)
    ball = jnp.concatenate([b1, b2, b3], axis=1).T

    r = np.arange(HW)
    j = (r // W // K) * Wo + (r % W) // K
    p_np = np.zeros((HW, 256), ml_dtypes.bfloat16)
    p_np[r, j] = 1.0
    pmat = jnp.asarray(p_np)

    G = 16
    x3 = x.reshape(N, Cin, HW)
    out = pl.pallas_call(
        functools.partial(_bottleneck_body, H=H, W=W, G=G, Cin=Cin, C2=C2),
        out_shape=jax.ShapeDtypeStruct((N, Cout, HoWo), jnp.float32),
        grid=(N // G,),
        in_specs=[
            pl.BlockSpec((G, Cin, HW), lambda i: (i, 0, 0)),
            pl.BlockSpec((C2, Cin + 9 * C2), lambda i: (0, 0)),
            pl.BlockSpec((Cout, C2 + Cin), lambda i: (0, 0)),
            pl.BlockSpec((2 * C2 + Cout, 1), lambda i: (0, 0)),
            pl.BlockSpec((HW, 256), lambda i: (0, 0)),
        ],
        out_specs=pl.BlockSpec((G, Cout, HoWo), lambda i: (i, 0, 0)),
        compiler_params=pltpu.CompilerParams(
            dimension_semantics=("parallel",),
            vmem_limit_bytes=64 * 1024 * 1024,
        ),
    )(x3, w12, w3t, ball, pmat)
    return out.reshape(N, Cout, Ho, Wo)


# speedup vs baseline: 1.0269x; 1.0269x over previous
import functools

import jax
import jax.numpy as jnp
import ml_dtypes
import numpy as np
from jax.experimental import pallas as pl
from jax.experimental.pallas import tpu as pltpu

SEG = 896


def _bottleneck_body(x_ref, w12_ref, w3_ref, b_ref, p_ref, o_ref, *,
                     H, W, G, Cin, C2):
    HW = H * W
    L = G * SEG
    w1_v = w12_ref[:, :Cin]
    w2_v = w12_ref[:, Cin:]
    b1_v = b_ref[:C2]
    b2_v = b_ref[C2:2 * C2]
    b3_v = b_ref[2 * C2:]
    lane = jax.lax.broadcasted_iota(jnp.int32, (1, L), 1)
    pos = lane % SEG
    wmod = pos % W
    seg_m = (pos < HW).astype(jnp.bfloat16)
    m_left = (wmod >= 1).astype(jnp.bfloat16)
    m_right = (wmod <= W - 2).astype(jnp.bfloat16)

    segs = [jnp.pad(x_ref[g].astype(jnp.bfloat16), ((0, 0), (0, SEG - HW)))
            for g in range(G)]
    x2 = jnp.concatenate(segs, axis=1)

    h1 = jnp.dot(w1_v, x2, preferred_element_type=jnp.float32)
    h1 = jnp.maximum(h1 + b1_v, 0.0)
    h1b = h1.astype(jnp.bfloat16) * seg_m

    hp1 = jnp.pad(h1b, ((0, 0), (1, 1)))
    base = jnp.concatenate(
        [hp1[:, 0:L] * m_left, h1b, hp1[:, 2:2 + L] * m_right],
        axis=0)
    bp = jnp.pad(base, ((0, 0), (W, W)))
    pat = jnp.concatenate(
        [bp[:, 0:L], base, bp[:, 2 * W:2 * W + L]], axis=0)
    h2 = jnp.dot(w2_v, pat, preferred_element_type=jnp.float32)
    h2 = jnp.maximum(h2 + b2_v, 0.0)
    h2b = h2.astype(jnp.bfloat16)

    CR = C2 + Cin
    big = jnp.concatenate(
        [h2b[:, g * SEG: g * SEG + HW] if k == 0 else
         x2[:, g * SEG: g * SEG + HW]
         for g in range(G) for k in (0, 1)], axis=0)
    po = jnp.dot(big, p_ref[...], preferred_element_type=jnp.float32)
    cat = jnp.concatenate(
        [po[g * CR:(g + 1) * CR].astype(jnp.bfloat16) for g in range(G)],
        axis=1)
    y = jnp.dot(w3_ref[...], cat, preferred_element_type=jnp.float32)
    y = jnp.maximum(y + b3_v, 0.0)
    HoWo = o_ref.shape[-1]
    for g in range(G):
        o_ref[g] = y[:, g * 256: g * 256 + HoWo]


def kernel(x, w1, b1, w2, b2, w3, wd, b3):
    N, Cin, H, W = x.shape
    C2 = w1.shape[1]
    Cout = w3.shape[1]
    K = 2
    Ho, Wo = H // K, W // K
    HW, HoWo = H * W, Ho * Wo

    w12 = jnp.concatenate(
        [w1.T, w2.transpose(2, 0, 1).reshape(C2, 9 * C2)],
        axis=1).astype(jnp.bfloat16)
    w3t = jnp.concatenate([w3, wd], axis=0).T.astype(jnp.bfloat16)
    ball = jnp.concatenate([b1, b2, b3], axis=1).T

    r = np.arange(HW)
    j = (r // W // K) * Wo + (r % W) // K
    p_np = np.zeros((HW, 256), ml_dtypes.bfloat16)
    p_np[r, j] = 1.0
    pmat = jnp.asarray(p_np)

    G = 8
    x3 = x.reshape(N, Cin, HW)
    out = pl.pallas_call(
        functools.partial(_bottleneck_body, H=H, W=W, G=G, Cin=Cin, C2=C2),
        out_shape=jax.ShapeDtypeStruct((N, Cout, HoWo), jnp.float32),
        grid=(N // G,),
        in_specs=[
            pl.BlockSpec((G, Cin, HW), lambda i: (i, 0, 0)),
            pl.BlockSpec((C2, Cin + 9 * C2), lambda i: (0, 0)),
            pl.BlockSpec((Cout, C2 + Cin), lambda i: (0, 0)),
            pl.BlockSpec((2 * C2 + Cout, 1), lambda i: (0, 0)),
            pl.BlockSpec((HW, 256), lambda i: (0, 0)),
        ],
        out_specs=pl.BlockSpec((G, Cout, HoWo), lambda i: (i, 0, 0)),
        compiler_params=pltpu.CompilerParams(
            dimension_semantics=("parallel",),
            vmem_limit_bytes=64 * 1024 * 1024,
        ),
    )(x3, w12, w3t, ball, pmat)
    return out.reshape(N, Cout, Ho, Wo)
